# baseline (speedup 1.0000x reference)
import functools

import jax
import jax.numpy as jnp
from jax import lax
from jax.experimental import pallas as pl
from jax.experimental.pallas import tpu as pltpu
from jax.experimental.pallas import tpu_sc as plsc

NC = 2
NS = 16
L = 16
CH = 80
SEC = 4000
NEG = -3.0e38


def _score_tc(M, a, E, D):
    BE = 4096

    def body(m_ref, a_ref, o_ref):
        o_ref[...] = jnp.sum(m_ref[...] * a_ref[...][None, :], axis=1)

    return pl.pallas_call(
        body,
        grid=(pl.cdiv(E, BE),),
        in_specs=[
            pl.BlockSpec((BE, D), lambda i: (i, 0)),
            pl.BlockSpec((D,), lambda i: (0,)),
        ],
        out_specs=pl.BlockSpec((BE,), lambda i: (i,)),
        out_shape=jax.ShapeDtypeStruct((E,), jnp.float32),
    )(M, a)


def _mega_sc(M, score, dest, src, rev, E, Np, D):
    H = D // 2
    EPC = E // NS
    ECH = EPC // CH
    NSEC = EPC // SEC
    SEG = Np // NS
    ZR = SEG // CH

    mesh = plsc.VectorSubcoreMesh(
        core_axis_name="c", subcore_axis_name="s",
        num_cores=NC, num_subcores=NS)

    @functools.partial(
        pl.kernel,
        out_type=(jax.ShapeDtypeStruct((E, D), jnp.float32),
                  jax.ShapeDtypeStruct((E, H), jnp.float32),
                  jax.ShapeDtypeStruct((E, H), jnp.float32)),
        mesh=mesh,
        compiler_params=pltpu.CompilerParams(
            needs_layout_passes=False, use_tc_tiling_on_sc=False),
        scratch_types=[
            pltpu.VMEM((SEC,), jnp.float32),
            pltpu.VMEM((SEC,), jnp.int32),
            pltpu.VMEM((SEC,), jnp.int32),
            pltpu.VMEM((Np,), jnp.float32),
            pltpu.VMEM((Np,), jnp.float32),
            pltpu.VMEM((SEG,), jnp.float32),
            pltpu.VMEM((SEG,), jnp.float32),
            pltpu.VMEM((CH,), jnp.float32),
            pltpu.VMEM((CH,), jnp.float32),
            pltpu.VMEM((CH,), jnp.int32),
            pltpu.VMEM((CH, H), jnp.float32),
            pltpu.VMEM((CH, H), jnp.float32),
            pltpu.VMEM((CH,), jnp.int32),
            pltpu.VMEM((CH,), jnp.int32),
            pltpu.VMEM((CH,), jnp.int32),
            pltpu.VMEM((CH,), jnp.int32),
            pltpu.VMEM((CH, H), jnp.float32),
            pltpu.VMEM((CH, H), jnp.float32),
            pltpu.VMEM((CH, H), jnp.float32),
            pltpu.VMEM((CH, H), jnp.float32),
            pltpu.VMEM((CH, H), jnp.float32),
            pltpu.VMEM((CH, H), jnp.float32),
            pltpu.VMEM_SHARED((NS, Np), jnp.float32),
            pltpu.VMEM_SHARED((Np, H), jnp.float32),
            pltpu.SemaphoreType.DMA,
            pltpu.SemaphoreType.DMA,
            pltpu.SemaphoreType.DMA,
            pltpu.SemaphoreType.DMA,
        ],
    )
    def k(m_hbm, score_hbm, dest_hbm, src_hbm, rev_hbm,
          out_hbm, wlo_hbm, whi_hbm,
          sc_sec, d_sec, rev_sec, smax, den, comb_v, tmp_v,
          sc_v, al_v, d_v, rows0, rows1, src_v, rev_v, src_v1, rev_v1,
          mv_v, wr_v, o_v, mv_v1, wr_v1, o_v1,
          stage, mv_s, sem0, sem1, semw0, semw1):
        cid = lax.axis_index("c")
        sid = lax.axis_index("s")
        n0 = sid * SEG
        e_base = sid * EPC

        @pl.loop(0, CH)
        def _(r):
            for g in range(H // L):
                o_v[r, pl.ds(g * L, L)] = jnp.zeros((L,), jnp.float32)

        @pl.loop(0, ZR)
        def _(i):
            pltpu.sync_copy(o_v, mv_s.at[pl.ds(n0 + i * CH, CH)])

        @pl.loop(0, Np // L)
        def _(i):
            den[pl.ds(i * L, L)] = jnp.full((L,), NEG, jnp.float32)

        @pl.loop(0, NSEC)
        def _(sec):
            ssl = pl.ds(e_base + sec * SEC, SEC)
            pltpu.sync_copy(score_hbm.at[ssl], sc_sec)
            pltpu.sync_copy(dest_hbm.at[ssl], d_sec)

            @pl.loop(0, SEC // L)
            def _(j):
                d = d_sec[pl.ds(j * L, L)]
                s = sc_sec[pl.ds(j * L, L)]
                cur = plsc.load_gather(den, [d])

                def cond(c):
                    return jnp.any(c)

                def body(c):
                    plsc.store_scatter(den, [d], s, mask=c)
                    return s > plsc.load_gather(den, [d])

                lax.while_loop(cond, body, s > cur)

        pltpu.sync_copy(den, stage.at[sid])
        plsc.subcore_barrier()
        pltpu.sync_copy(stage.at[0, pl.ds(n0, SEG)], comb_v)
        for r in range(1, NS):
            pltpu.sync_copy(stage.at[r, pl.ds(n0, SEG)], tmp_v)

            @pl.loop(0, SEG // L)
            def _(i):
                sl = pl.ds(i * L, L)
                comb_v[sl] = jnp.maximum(comb_v[sl], tmp_v[sl])

        plsc.subcore_barrier()
        pltpu.sync_copy(comb_v, stage.at[0, pl.ds(n0, SEG)])
        plsc.subcore_barrier()
        pltpu.sync_copy(stage.at[0], smax)
        plsc.subcore_barrier()

        @pl.loop(0, Np // L)
        def _(i):
            den[pl.ds(i * L, L)] = jnp.zeros((L,), jnp.float32)

        @pl.loop(0, NSEC)
        def _(sec):
            ssl = pl.ds(e_base + sec * SEC, SEC)
            pltpu.sync_copy(score_hbm.at[ssl], sc_sec)
            pltpu.sync_copy(dest_hbm.at[ssl], d_sec)

            @pl.loop(0, SEC // L)
            def _(j):
                d = d_sec[pl.ds(j * L, L)]
                s = sc_sec[pl.ds(j * L, L)]
                sm = plsc.load_gather(smax, [d])
                plsc.addupdate_scatter(den, [d], jnp.exp(s - sm))

        pltpu.sync_copy(den, stage.at[sid])
        plsc.subcore_barrier()
        pltpu.sync_copy(stage.at[0, pl.ds(n0, SEG)], comb_v)
        for r in range(1, NS):
            pltpu.sync_copy(stage.at[r, pl.ds(n0, SEG)], tmp_v)

            @pl.loop(0, SEG // L)
            def _(i):
                sl = pl.ds(i * L, L)
                comb_v[sl] = comb_v[sl] + tmp_v[sl]

        plsc.subcore_barrier()
        pltpu.sync_copy(comb_v, stage.at[0, pl.ds(n0, SEG)])
        plsc.subcore_barrier()
        pltpu.sync_copy(stage.at[0], den)

        CPS = SEC // CH

        def a_load(sec, k, buf, sem):
            sl = pl.ds(e_base + sec * SEC + k * CH, CH)

            @pl.when(cid == 0)
            def _():
                pltpu.async_copy(m_hbm.at[sl, pl.ds(0, H)], buf, sem)

            @pl.when(cid == 1)
            def _():
                pltpu.async_copy(m_hbm.at[sl, pl.ds(H, H)], buf, sem)

        def a_wait_load(buf, sem):
            pltpu.make_async_copy(
                m_hbm.at[pl.ds(0, CH), pl.ds(0, H)], buf, sem).wait()

        def a_alpha(k):
            off = k * CH

            @pl.loop(0, CH // L)
            def _(jj):
                d = d_sec[pl.ds(off + jj * L, L)]
                s = sc_sec[pl.ds(off + jj * L, L)]
                sm = plsc.load_gather(smax, [d])
                dn = plsc.load_gather(den, [d])
                al_v[pl.ds(jj * L, L)] = jnp.exp(s - sm) / (dn + 1e-16)
                d_v[pl.ds(jj * L, L)] = d

        def a_proc(sec, k, buf, semw):
            sl = pl.ds(e_base + sec * SEC + k * CH, CH)

            @pl.loop(0, CH // L)
            def _(jj):
                alv = al_v[pl.ds(jj * L, L)]
                for r16 in range(L):
                    a_s = alv[r16]
                    row = jj * L + r16
                    for g in range(H // L):
                        rsl = pl.ds(g * L, L)
                        buf[row, rsl] = buf[row, rsl] * a_s

            @pl.when(cid == 0)
            def _():
                pltpu.async_copy(buf, wlo_hbm.at[sl], semw)

            @pl.when(cid == 1)
            def _():
                pltpu.async_copy(buf, whi_hbm.at[sl], semw)

            pltpu.sync_copy(buf, mv_s.at[d_v], add=True)
            pltpu.make_async_copy(buf, wlo_hbm.at[pl.ds(0, CH)], semw).wait()

        @pl.loop(0, NSEC)
        def _(sec):
            ssl = pl.ds(e_base + sec * SEC, SEC)
            pltpu.sync_copy(score_hbm.at[ssl], sc_sec)
            pltpu.sync_copy(dest_hbm.at[ssl], d_sec)
            a_load(sec, 0, rows0, sem0)

            @pl.loop(0, CPS // 2)
            def _(kk):
                k0 = kk * 2
                a_load(sec, k0 + 1, rows1, sem1)
                a_alpha(k0)
                a_wait_load(rows0, sem0)
                a_proc(sec, k0, rows0, semw0)

                @pl.when(k0 + 2 < CPS)
                def _():
                    a_load(sec, k0 + 2, rows0, sem0)

                a_alpha(k0 + 1)
                a_wait_load(rows1, sem1)
                a_proc(sec, k0 + 1, rows1, semw1)

        plsc.subcore_barrier()

        def b_fill(k, sv, rv):
            @pl.loop(0, CH // L)
            def _(t):
                sv[pl.ds(t * L, L)] = d_sec[pl.ds(k * CH + t * L, L)]
                rv[pl.ds(t * L, L)] = rev_sec[pl.ds(k * CH + t * L, L)]

        def b_prep(k, sv, rv, mvb, wrb, sem):
            iv = pl.ds(k * CH, CH)

            @pl.when(cid == 0)
            def _():
                pltpu.async_copy(wlo_hbm.at[rev_sec.at[iv]], wrb, sem)

            @pl.when(cid == 1)
            def _():
                pltpu.async_copy(whi_hbm.at[rev_sec.at[iv]], wrb, sem)

            pltpu.sync_copy(mv_s.at[d_sec.at[iv]], mvb)

        def b_out(sec, k, kk, mvb, wrb, ob, semw, sem):
            pltpu.make_async_copy(wlo_hbm.at[pl.ds(0, CH)], wrb, sem).wait()

            @pl.when(kk > 0)
            def _():
                pltpu.make_async_copy(
                    ob, out_hbm.at[pl.ds(0, CH), pl.ds(0, H)], semw).wait()

            @pl.loop(0, CH, unroll=8)
            def _(r):
                for g in range(H // L):
                    csl = pl.ds(g * L, L)
                    ob[r, csl] = mvb[r, csl] - wrb[r, csl]

            sl = pl.ds(e_base + sec * SEC + k * CH, CH)

            @pl.when(cid == 0)
            def _():
                pltpu.async_copy(ob, out_hbm.at[sl, pl.ds(0, H)], semw)

            @pl.when(cid == 1)
            def _():
                pltpu.async_copy(ob, out_hbm.at[sl, pl.ds(H, H)], semw)

        @pl.loop(0, NSEC)
        def _(sec):
            ssl = pl.ds(e_base + sec * SEC, SEC)
            pltpu.sync_copy(src_hbm.at[ssl], d_sec)
            pltpu.sync_copy(rev_hbm.at[ssl], rev_sec)
            b_prep(0, src_v, rev_v, mv_v, wr_v, sem0)

            @pl.loop(0, CPS // 2)
            def _(kk):
                k0 = kk * 2
                b_prep(k0 + 1, src_v1, rev_v1, mv_v1, wr_v1, sem1)
                b_out(sec, k0, kk, mv_v, wr_v, o_v, semw0, sem0)

                @pl.when(k0 + 2 < CPS)
                def _():
                    b_prep(k0 + 2, src_v, rev_v, mv_v, wr_v, sem0)

                b_out(sec, k0 + 1, kk, mv_v1, wr_v1, o_v1, semw1, sem1)

            pltpu.make_async_copy(
                o_v, out_hbm.at[pl.ds(0, CH), pl.ds(0, H)], semw0).wait()
            pltpu.make_async_copy(
                o_v1, out_hbm.at[pl.ds(0, CH), pl.ds(0, H)], semw1).wait()

    return k(M, score, dest, src, rev)


def kernel(M, edge_index, rev_index, dim_size, a):
    E, D = M.shape
    Np = 10240
    src = edge_index[0]
    dest = edge_index[1]
    score = _score_tc(M, a, E, D)
    out, _, _ = _mega_sc(M, score, dest, src, rev_index, E, Np, D)
    return out


# speedup vs baseline: 1.0008x; 1.0008x over previous
import functools

import jax
import jax.numpy as jnp
from jax import lax
from jax.experimental import pallas as pl
from jax.experimental.pallas import tpu as pltpu
from jax.experimental.pallas import tpu_sc as plsc

NC = 2
NS = 16
L = 16
CH = 80
SEC = 4000
NEG = -3.0e38


def _score_tc(M, a, E, D):
    BE = 4096

    def body(m_ref, a_ref, o_ref):
        o_ref[...] = jnp.sum(m_ref[...] * a_ref[...][None, :], axis=1)

    return pl.pallas_call(
        body,
        grid=(pl.cdiv(E, BE),),
        in_specs=[
            pl.BlockSpec((BE, D), lambda i: (i, 0)),
            pl.BlockSpec((D,), lambda i: (0,)),
        ],
        out_specs=pl.BlockSpec((BE,), lambda i: (i,)),
        out_shape=jax.ShapeDtypeStruct((E,), jnp.float32),
    )(M, a)


def _mega_sc(M, score, dest, src, rev, E, Np, D):
    H = D // 2
    EPC = E // NS
    ECH = EPC // CH
    NSEC = EPC // SEC
    SEG = Np // NS
    ZR = SEG // CH

    mesh = plsc.VectorSubcoreMesh(
        core_axis_name="c", subcore_axis_name="s",
        num_cores=NC, num_subcores=NS)

    @functools.partial(
        pl.kernel,
        out_type=(jax.ShapeDtypeStruct((E, D), jnp.float32),
                  jax.ShapeDtypeStruct((E, H), jnp.float32),
                  jax.ShapeDtypeStruct((E, H), jnp.float32)),
        mesh=mesh,
        compiler_params=pltpu.CompilerParams(
            needs_layout_passes=False, use_tc_tiling_on_sc=False),
        scratch_types=[
            pltpu.VMEM((SEC,), jnp.float32),
            pltpu.VMEM((SEC,), jnp.int32),
            pltpu.VMEM((SEC,), jnp.int32),
            pltpu.VMEM((Np,), jnp.float32),
            pltpu.VMEM((Np,), jnp.float32),
            pltpu.VMEM((SEG,), jnp.float32),
            pltpu.VMEM((SEG,), jnp.float32),
            pltpu.VMEM((CH,), jnp.float32),
            pltpu.VMEM((CH,), jnp.float32),
            pltpu.VMEM((CH,), jnp.int32),
            pltpu.VMEM((CH, H), jnp.float32),
            pltpu.VMEM((CH, H), jnp.float32),
            pltpu.VMEM((CH,), jnp.int32),
            pltpu.VMEM((CH,), jnp.int32),
            pltpu.VMEM((CH,), jnp.int32),
            pltpu.VMEM((CH,), jnp.int32),
            pltpu.VMEM((CH, H), jnp.float32),
            pltpu.VMEM((CH, H), jnp.float32),
            pltpu.VMEM((CH, H), jnp.float32),
            pltpu.VMEM((CH, H), jnp.float32),
            pltpu.VMEM((CH, H), jnp.float32),
            pltpu.VMEM((CH, H), jnp.float32),
            pltpu.VMEM_SHARED((NS, Np), jnp.float32),
            pltpu.VMEM_SHARED((Np, H), jnp.float32),
            pltpu.SemaphoreType.DMA,
            pltpu.SemaphoreType.DMA,
            pltpu.SemaphoreType.DMA,
            pltpu.SemaphoreType.DMA,
        ],
    )
    def k(m_hbm, score_hbm, dest_hbm, src_hbm, rev_hbm,
          out_hbm, wlo_hbm, whi_hbm,
          sc_sec, d_sec, rev_sec, smax, den, comb_v, tmp_v,
          sc_v, al_v, d_v, rows0, rows1, src_v, rev_v, src_v1, rev_v1,
          mv_v, wr_v, o_v, mv_v1, wr_v1, o_v1,
          stage, mv_s, sem0, sem1, semw0, semw1):
        cid = lax.axis_index("c")
        sid = lax.axis_index("s")
        n0 = sid * SEG
        e_base = sid * EPC

        @pl.loop(0, CH)
        def _(r):
            for g in range(H // L):
                o_v[r, pl.ds(g * L, L)] = jnp.zeros((L,), jnp.float32)

        @pl.loop(0, ZR)
        def _(i):
            pltpu.sync_copy(o_v, mv_s.at[pl.ds(n0 + i * CH, CH)])

        @pl.loop(0, Np // L)
        def _(i):
            den[pl.ds(i * L, L)] = jnp.full((L,), NEG, jnp.float32)

        @pl.loop(0, NSEC)
        def _(sec):
            ssl = pl.ds(e_base + sec * SEC, SEC)
            pltpu.sync_copy(score_hbm.at[ssl], sc_sec)
            pltpu.sync_copy(dest_hbm.at[ssl], d_sec)

            @pl.loop(0, SEC // L)
            def _(j):
                d = d_sec[pl.ds(j * L, L)]
                s = sc_sec[pl.ds(j * L, L)]
                cur = plsc.load_gather(den, [d])

                def cond(c):
                    return jnp.any(c)

                def body(c):
                    plsc.store_scatter(den, [d], s, mask=c)
                    return s > plsc.load_gather(den, [d])

                lax.while_loop(cond, body, s > cur)

        pltpu.sync_copy(den, stage.at[sid])
        plsc.subcore_barrier()
        pltpu.sync_copy(stage.at[0, pl.ds(n0, SEG)], comb_v)
        for r in range(1, NS):
            pltpu.sync_copy(stage.at[r, pl.ds(n0, SEG)], tmp_v)

            @pl.loop(0, SEG // L)
            def _(i):
                sl = pl.ds(i * L, L)
                comb_v[sl] = jnp.maximum(comb_v[sl], tmp_v[sl])

        plsc.subcore_barrier()
        pltpu.sync_copy(comb_v, stage.at[0, pl.ds(n0, SEG)])
        plsc.subcore_barrier()
        pltpu.sync_copy(stage.at[0], smax)
        plsc.subcore_barrier()

        @pl.loop(0, Np // L)
        def _(i):
            den[pl.ds(i * L, L)] = jnp.zeros((L,), jnp.float32)

        @pl.loop(0, NSEC)
        def _(sec):
            ssl = pl.ds(e_base + sec * SEC, SEC)
            pltpu.sync_copy(score_hbm.at[ssl], sc_sec)
            pltpu.sync_copy(dest_hbm.at[ssl], d_sec)

            @pl.loop(0, SEC // L)
            def _(j):
                d = d_sec[pl.ds(j * L, L)]
                s = sc_sec[pl.ds(j * L, L)]
                sm = plsc.load_gather(smax, [d])
                plsc.addupdate_scatter(den, [d], jnp.exp(s - sm))

        pltpu.sync_copy(den, stage.at[sid])
        plsc.subcore_barrier()
        pltpu.sync_copy(stage.at[0, pl.ds(n0, SEG)], comb_v)
        for r in range(1, NS):
            pltpu.sync_copy(stage.at[r, pl.ds(n0, SEG)], tmp_v)

            @pl.loop(0, SEG // L)
            def _(i):
                sl = pl.ds(i * L, L)
                comb_v[sl] = comb_v[sl] + tmp_v[sl]

        plsc.subcore_barrier()
        pltpu.sync_copy(comb_v, stage.at[0, pl.ds(n0, SEG)])
        plsc.subcore_barrier()
        pltpu.sync_copy(stage.at[0], den)

        CPS = SEC // CH

        def a_load(sec, k, buf, sem):
            sl = pl.ds(e_base + sec * SEC + k * CH, CH)

            @pl.when(cid == 0)
            def _():
                pltpu.async_copy(m_hbm.at[sl, pl.ds(0, H)], buf, sem)

            @pl.when(cid == 1)
            def _():
                pltpu.async_copy(m_hbm.at[sl, pl.ds(H, H)], buf, sem)

        def a_wait_load(buf, sem):
            pltpu.make_async_copy(
                m_hbm.at[pl.ds(0, CH), pl.ds(0, H)], buf, sem).wait()

        def a_alpha(k):
            off = k * CH

            @pl.loop(0, CH // L)
            def _(jj):
                d = d_sec[pl.ds(off + jj * L, L)]
                s = sc_sec[pl.ds(off + jj * L, L)]
                sm = plsc.load_gather(smax, [d])
                dn = plsc.load_gather(den, [d])
                al_v[pl.ds(jj * L, L)] = jnp.exp(s - sm) / (dn + 1e-16)
                d_v[pl.ds(jj * L, L)] = d

        def a_proc(sec, k, buf, semw):
            sl = pl.ds(e_base + sec * SEC + k * CH, CH)

            @pl.loop(0, CH // L)
            def _(jj):
                alv = al_v[pl.ds(jj * L, L)]
                for r16 in range(L):
                    a_s = alv[r16]
                    row = jj * L + r16
                    for g in range(H // L):
                        rsl = pl.ds(g * L, L)
                        buf[row, rsl] = buf[row, rsl] * a_s

            @pl.when(cid == 0)
            def _():
                pltpu.async_copy(buf, wlo_hbm.at[sl], semw)

            @pl.when(cid == 1)
            def _():
                pltpu.async_copy(buf, whi_hbm.at[sl], semw)

            pltpu.sync_copy(buf, mv_s.at[d_v], add=True)
            pltpu.make_async_copy(buf, wlo_hbm.at[pl.ds(0, CH)], semw).wait()

        @pl.loop(0, NSEC)
        def _(sec):
            ssl = pl.ds(e_base + sec * SEC, SEC)
            pltpu.sync_copy(score_hbm.at[ssl], sc_sec)
            pltpu.sync_copy(dest_hbm.at[ssl], d_sec)
            a_load(sec, 0, rows0, sem0)

            @pl.loop(0, CPS // 2)
            def _(kk):
                k0 = kk * 2
                a_load(sec, k0 + 1, rows1, sem1)
                a_alpha(k0)
                a_wait_load(rows0, sem0)
                a_proc(sec, k0, rows0, semw0)

                @pl.when(k0 + 2 < CPS)
                def _():
                    a_load(sec, k0 + 2, rows0, sem0)

                a_alpha(k0 + 1)
                a_wait_load(rows1, sem1)
                a_proc(sec, k0 + 1, rows1, semw1)

        plsc.subcore_barrier()

        def b_fill(k, sv, rv):
            @pl.loop(0, CH // L)
            def _(t):
                sv[pl.ds(t * L, L)] = d_sec[pl.ds(k * CH + t * L, L)]
                rv[pl.ds(t * L, L)] = rev_sec[pl.ds(k * CH + t * L, L)]

        def b_prep(k, sv, rv, mvb, wrb, sem):
            iv = pl.ds(k * CH, CH)

            @pl.when(cid == 0)
            def _():
                pltpu.async_copy(wlo_hbm.at[rev_sec.at[iv]], wrb, sem)

            @pl.when(cid == 1)
            def _():
                pltpu.async_copy(whi_hbm.at[rev_sec.at[iv]], wrb, sem)

            pltpu.sync_copy(mv_s.at[d_sec.at[iv]], mvb)

        def b_out(sec, k, kk, mvb, wrb, ob, semw, sem):
            pltpu.make_async_copy(wlo_hbm.at[pl.ds(0, CH)], wrb, sem).wait()

            @pl.when(kk > 0)
            def _():
                pltpu.make_async_copy(
                    ob, out_hbm.at[pl.ds(0, CH), pl.ds(0, H)], semw).wait()

            @pl.loop(0, CH, unroll=4)
            def _(r):
                for g in range(H // L):
                    csl = pl.ds(g * L, L)
                    ob[r, csl] = mvb[r, csl] - wrb[r, csl]

            sl = pl.ds(e_base + sec * SEC + k * CH, CH)

            @pl.when(cid == 0)
            def _():
                pltpu.async_copy(ob, out_hbm.at[sl, pl.ds(0, H)], semw)

            @pl.when(cid == 1)
            def _():
                pltpu.async_copy(ob, out_hbm.at[sl, pl.ds(H, H)], semw)

        @pl.loop(0, NSEC)
        def _(sec):
            ssl = pl.ds(e_base + sec * SEC, SEC)
            pltpu.sync_copy(src_hbm.at[ssl], d_sec)
            pltpu.sync_copy(rev_hbm.at[ssl], rev_sec)
            b_prep(0, src_v, rev_v, mv_v, wr_v, sem0)

            @pl.loop(0, CPS // 2)
            def _(kk):
                k0 = kk * 2
                b_prep(k0 + 1, src_v1, rev_v1, mv_v1, wr_v1, sem1)
                b_out(sec, k0, kk, mv_v, wr_v, o_v, semw0, sem0)

                @pl.when(k0 + 2 < CPS)
                def _():
                    b_prep(k0 + 2, src_v, rev_v, mv_v, wr_v, sem0)

                b_out(sec, k0 + 1, kk, mv_v1, wr_v1, o_v1, semw1, sem1)

            pltpu.make_async_copy(
                o_v, out_hbm.at[pl.ds(0, CH), pl.ds(0, H)], semw0).wait()
            pltpu.make_async_copy(
                o_v1, out_hbm.at[pl.ds(0, CH), pl.ds(0, H)], semw1).wait()

    return k(M, score, dest, src, rev)


def kernel(M, edge_index, rev_index, dim_size, a):
    E, D = M.shape
    Np = 10240
    src = edge_index[0]
    dest = edge_index[1]
    score = _score_tc(M, a, E, D)
    out, _, _ = _mega_sc(M, score, dest, src, rev_index, E, Np, D)
    return out


# speedup vs baseline: 1.0014x; 1.0006x over previous
import functools

import jax
import jax.numpy as jnp
from jax import lax
from jax.experimental import pallas as pl
from jax.experimental.pallas import tpu as pltpu
from jax.experimental.pallas import tpu_sc as plsc

NC = 2
NS = 16
L = 16
CH = 80
SEC = 4000
NEG = -3.0e38


def _score_tc(M, a, E, D):
    BE = 4096

    def body(m_ref, a_ref, o_ref):
        o_ref[...] = jnp.sum(m_ref[...] * a_ref[...][None, :], axis=1)

    return pl.pallas_call(
        body,
        grid=(pl.cdiv(E, BE),),
        in_specs=[
            pl.BlockSpec((BE, D), lambda i: (i, 0)),
            pl.BlockSpec((D,), lambda i: (0,)),
        ],
        out_specs=pl.BlockSpec((BE,), lambda i: (i,)),
        out_shape=jax.ShapeDtypeStruct((E,), jnp.float32),
    )(M, a)


def _mega_sc(M, score, dest, src, rev, E, Np, D):
    H = D // 2
    EPC = E // NS
    ECH = EPC // CH
    NSEC = EPC // SEC
    SEG = Np // NS
    ZR = SEG // CH

    mesh = plsc.VectorSubcoreMesh(
        core_axis_name="c", subcore_axis_name="s",
        num_cores=NC, num_subcores=NS)

    @functools.partial(
        pl.kernel,
        out_type=(jax.ShapeDtypeStruct((E, D), jnp.float32),
                  jax.ShapeDtypeStruct((E, H), jnp.float32),
                  jax.ShapeDtypeStruct((E, H), jnp.float32)),
        mesh=mesh,
        compiler_params=pltpu.CompilerParams(
            needs_layout_passes=False, use_tc_tiling_on_sc=False),
        scratch_types=[
            pltpu.VMEM((SEC,), jnp.float32),
            pltpu.VMEM((SEC,), jnp.int32),
            pltpu.VMEM((SEC,), jnp.int32),
            pltpu.VMEM((Np,), jnp.float32),
            pltpu.VMEM((Np,), jnp.float32),
            pltpu.VMEM((SEG,), jnp.float32),
            pltpu.VMEM((SEG,), jnp.float32),
            pltpu.VMEM((CH,), jnp.float32),
            pltpu.VMEM((CH,), jnp.int32),
            pltpu.VMEM((CH, H), jnp.float32),
            pltpu.VMEM((CH, H), jnp.float32),
            pltpu.VMEM((CH, H), jnp.float32),
            pltpu.VMEM((CH, H), jnp.float32),
            pltpu.VMEM((CH, H), jnp.float32),
            pltpu.VMEM((CH, H), jnp.float32),
            pltpu.VMEM((CH, H), jnp.float32),
            pltpu.VMEM((CH, H), jnp.float32),
            pltpu.VMEM_SHARED((NS, Np), jnp.float32),
            pltpu.VMEM_SHARED((Np, H), jnp.float32),
            pltpu.SemaphoreType.DMA,
            pltpu.SemaphoreType.DMA,
            pltpu.SemaphoreType.DMA,
            pltpu.SemaphoreType.DMA,
        ],
    )
    def k(m_hbm, score_hbm, dest_hbm, src_hbm, rev_hbm,
          out_hbm, wlo_hbm, whi_hbm,
          sc_sec, d_sec, rev_sec, smax, den, comb_v, tmp_v,
          al_v, d_v, rows0, rows1,
          mv_v, wr_v, o_v, mv_v1, wr_v1, o_v1,
          stage, mv_s, sem0, sem1, semw0, semw1):
        cid = lax.axis_index("c")
        sid = lax.axis_index("s")
        n0 = sid * SEG
        e_base = sid * EPC

        @pl.loop(0, CH)
        def _(r):
            for g in range(H // L):
                o_v[r, pl.ds(g * L, L)] = jnp.zeros((L,), jnp.float32)

        @pl.loop(0, ZR)
        def _(i):
            pltpu.sync_copy(o_v, mv_s.at[pl.ds(n0 + i * CH, CH)])

        @pl.loop(0, Np // L)
        def _(i):
            den[pl.ds(i * L, L)] = jnp.full((L,), NEG, jnp.float32)

        @pl.loop(0, NSEC)
        def _(sec):
            ssl = pl.ds(e_base + sec * SEC, SEC)
            pltpu.sync_copy(score_hbm.at[ssl], sc_sec)
            pltpu.sync_copy(dest_hbm.at[ssl], d_sec)

            @pl.loop(0, SEC // L)
            def _(j):
                d = d_sec[pl.ds(j * L, L)]
                s = sc_sec[pl.ds(j * L, L)]
                cur = plsc.load_gather(den, [d])

                def cond(c):
                    return jnp.any(c)

                def body(c):
                    plsc.store_scatter(den, [d], s, mask=c)
                    return s > plsc.load_gather(den, [d])

                lax.while_loop(cond, body, s > cur)

        pltpu.sync_copy(den, stage.at[sid])
        plsc.subcore_barrier()
        pltpu.sync_copy(stage.at[0, pl.ds(n0, SEG)], comb_v)
        for r in range(1, NS):
            pltpu.sync_copy(stage.at[r, pl.ds(n0, SEG)], tmp_v)

            @pl.loop(0, SEG // L)
            def _(i):
                sl = pl.ds(i * L, L)
                comb_v[sl] = jnp.maximum(comb_v[sl], tmp_v[sl])

        plsc.subcore_barrier()
        pltpu.sync_copy(comb_v, stage.at[0, pl.ds(n0, SEG)])
        plsc.subcore_barrier()
        pltpu.sync_copy(stage.at[0], smax)
        plsc.subcore_barrier()

        @pl.loop(0, Np // L)
        def _(i):
            den[pl.ds(i * L, L)] = jnp.zeros((L,), jnp.float32)

        @pl.loop(0, NSEC)
        def _(sec):
            ssl = pl.ds(e_base + sec * SEC, SEC)
            pltpu.sync_copy(score_hbm.at[ssl], sc_sec)
            pltpu.sync_copy(dest_hbm.at[ssl], d_sec)

            @pl.loop(0, SEC // L)
            def _(j):
                d = d_sec[pl.ds(j * L, L)]
                s = sc_sec[pl.ds(j * L, L)]
                sm = plsc.load_gather(smax, [d])
                plsc.addupdate_scatter(den, [d], jnp.exp(s - sm))

        pltpu.sync_copy(den, stage.at[sid])
        plsc.subcore_barrier()
        pltpu.sync_copy(stage.at[0, pl.ds(n0, SEG)], comb_v)
        for r in range(1, NS):
            pltpu.sync_copy(stage.at[r, pl.ds(n0, SEG)], tmp_v)

            @pl.loop(0, SEG // L)
            def _(i):
                sl = pl.ds(i * L, L)
                comb_v[sl] = comb_v[sl] + tmp_v[sl]

        plsc.subcore_barrier()
        pltpu.sync_copy(comb_v, stage.at[0, pl.ds(n0, SEG)])
        plsc.subcore_barrier()
        pltpu.sync_copy(stage.at[0], den)

        CPS = SEC // CH

        def a_load(sec, k, buf, sem):
            sl = pl.ds(e_base + sec * SEC + k * CH, CH)

            @pl.when(cid == 0)
            def _():
                pltpu.async_copy(m_hbm.at[sl, pl.ds(0, H)], buf, sem)

            @pl.when(cid == 1)
            def _():
                pltpu.async_copy(m_hbm.at[sl, pl.ds(H, H)], buf, sem)

        def a_wait_load(buf, sem):
            pltpu.make_async_copy(
                m_hbm.at[pl.ds(0, CH), pl.ds(0, H)], buf, sem).wait()

        def a_alpha(k):
            off = k * CH

            @pl.loop(0, CH // L)
            def _(jj):
                d = d_sec[pl.ds(off + jj * L, L)]
                s = sc_sec[pl.ds(off + jj * L, L)]
                sm = plsc.load_gather(smax, [d])
                dn = plsc.load_gather(den, [d])
                al_v[pl.ds(jj * L, L)] = jnp.exp(s - sm) / (dn + 1e-16)
                d_v[pl.ds(jj * L, L)] = d

        def a_proc(sec, k, buf, semw):
            sl = pl.ds(e_base + sec * SEC + k * CH, CH)

            @pl.loop(0, CH // L)
            def _(jj):
                alv = al_v[pl.ds(jj * L, L)]
                for r16 in range(L):
                    a_s = alv[r16]
                    row = jj * L + r16
                    for g in range(H // L):
                        rsl = pl.ds(g * L, L)
                        buf[row, rsl] = buf[row, rsl] * a_s

            @pl.when(cid == 0)
            def _():
                pltpu.async_copy(buf, wlo_hbm.at[sl], semw)

            @pl.when(cid == 1)
            def _():
                pltpu.async_copy(buf, whi_hbm.at[sl], semw)

            pltpu.sync_copy(buf, mv_s.at[d_v], add=True)
            pltpu.make_async_copy(buf, wlo_hbm.at[pl.ds(0, CH)], semw).wait()

        @pl.loop(0, NSEC)
        def _(sec):
            ssl = pl.ds(e_base + sec * SEC, SEC)
            pltpu.sync_copy(score_hbm.at[ssl], sc_sec)
            pltpu.sync_copy(dest_hbm.at[ssl], d_sec)
            a_load(sec, 0, rows0, sem0)

            @pl.loop(0, CPS // 2)
            def _(kk):
                k0 = kk * 2
                a_load(sec, k0 + 1, rows1, sem1)
                a_alpha(k0)
                a_wait_load(rows0, sem0)
                a_proc(sec, k0, rows0, semw0)

                @pl.when(k0 + 2 < CPS)
                def _():
                    a_load(sec, k0 + 2, rows0, sem0)

                a_alpha(k0 + 1)
                a_wait_load(rows1, sem1)
                a_proc(sec, k0 + 1, rows1, semw1)

        plsc.subcore_barrier()

        def b_prep(k, mvb, wrb, sem):
            iv = pl.ds(k * CH, CH)

            @pl.when(cid == 0)
            def _():
                pltpu.async_copy(wlo_hbm.at[rev_sec.at[iv]], wrb, sem)

            @pl.when(cid == 1)
            def _():
                pltpu.async_copy(whi_hbm.at[rev_sec.at[iv]], wrb, sem)

            pltpu.sync_copy(mv_s.at[d_sec.at[iv]], mvb)

        def b_out(sec, k, kk, mvb, wrb, ob, semw, sem):
            pltpu.make_async_copy(wlo_hbm.at[pl.ds(0, CH)], wrb, sem).wait()

            @pl.when(kk > 0)
            def _():
                pltpu.make_async_copy(
                    ob, out_hbm.at[pl.ds(0, CH), pl.ds(0, H)], semw).wait()

            @pl.loop(0, CH, unroll=4)
            def _(r):
                for g in range(H // L):
                    csl = pl.ds(g * L, L)
                    ob[r, csl] = mvb[r, csl] - wrb[r, csl]

            sl = pl.ds(e_base + sec * SEC + k * CH, CH)

            @pl.when(cid == 0)
            def _():
                pltpu.async_copy(ob, out_hbm.at[sl, pl.ds(0, H)], semw)

            @pl.when(cid == 1)
            def _():
                pltpu.async_copy(ob, out_hbm.at[sl, pl.ds(H, H)], semw)

        @pl.loop(0, NSEC)
        def _(sec):
            ssl = pl.ds(e_base + sec * SEC, SEC)
            pltpu.sync_copy(src_hbm.at[ssl], d_sec)
            pltpu.sync_copy(rev_hbm.at[ssl], rev_sec)
            b_prep(0, mv_v, wr_v, sem0)

            @pl.loop(0, CPS // 2)
            def _(kk):
                k0 = kk * 2
                b_prep(k0 + 1, mv_v1, wr_v1, sem1)
                b_out(sec, k0, kk, mv_v, wr_v, o_v, semw0, sem0)

                @pl.when(k0 + 2 < CPS)
                def _():
                    b_prep(k0 + 2, mv_v, wr_v, sem0)

                b_out(sec, k0 + 1, kk, mv_v1, wr_v1, o_v1, semw1, sem1)

            pltpu.make_async_copy(
                o_v, out_hbm.at[pl.ds(0, CH), pl.ds(0, H)], semw0).wait()
            pltpu.make_async_copy(
                o_v1, out_hbm.at[pl.ds(0, CH), pl.ds(0, H)], semw1).wait()

    return k(M, score, dest, src, rev)


def kernel(M, edge_index, rev_index, dim_size, a):
    E, D = M.shape
    Np = 10240
    src = edge_index[0]
    dest = edge_index[1]
    score = _score_tc(M, a, E, D)
    out, _, _ = _mega_sc(M, score, dest, src, rev_index, E, Np, D)
    return out
